# Initial kernel scaffold; baseline (speedup 1.0000x reference)
import jax
import jax.numpy as jnp
from jax.experimental import pallas as pl


def kernel(points, raw_scale, raw_exponents, raw_rotation, raw_tapering, translation):
    raise NotImplementedError("write your pallas kernel here")


# speedup vs baseline: 3.7921x; 3.7921x over previous
import functools

import jax
import jax.numpy as jnp
from jax.experimental import pallas as pl
from jax.experimental.pallas import tpu as pltpu

_MINE, _MAXE = 0.1, 1.9
_TRUNC = 0.1
_EPS = 1e-6
_LOG2E = 1.4426950408889634


def _signclamp(v):
    return jnp.where(v > 0, 1.0, -1.0) * jnp.maximum(jnp.abs(v), _EPS)


def _sdf_block_kernel(points_ref, scale_ref, exps_ref, rot_ref, taper_ref,
                      trans_ref, out_ref):
    scale = jnp.exp(scale_ref[...]) + 1e-6
    inv_s = 1.0 / scale
    isx = inv_s[:, 0:1]
    isy = inv_s[:, 1:2]
    isz = inv_s[:, 2:3]

    e = jax.nn.sigmoid(exps_ref[...]) * (_MAXE - _MINE) + _MINE
    e1 = e[:, 0:1]
    e2 = e[:, 1:2]
    p2 = 2.0 / e2
    p21 = e2 / e1
    p1 = 2.0 / e1
    ph = -0.5 * e1

    q = rot_ref[...]
    q = q / (jnp.sqrt(jnp.sum(q * q, axis=-1, keepdims=True)) + 1e-12)
    qw = q[:, 0:1]
    qx = q[:, 1:2]
    qy = q[:, 2:3]
    qz = q[:, 3:4]
    r00 = 1 - 2 * (qy * qy + qz * qz)
    r01 = 2 * (qx * qy - qw * qz)
    r02 = 2 * (qx * qz + qw * qy)
    r10 = 2 * (qx * qy + qw * qz)
    r11 = 1 - 2 * (qx * qx + qz * qz)
    r12 = 2 * (qy * qz - qw * qx)
    r20 = 2 * (qx * qz - qw * qy)
    r21 = 2 * (qy * qz + qw * qx)
    r22 = 1 - 2 * (qx * qx + qy * qy)

    taper = jnp.tanh(taper_ref[...])
    cx = taper[:, 0:1] * isz
    cy = taper[:, 1:2] * isz

    tx = trans_ref[:, 0:1]
    ty = trans_ref[:, 1:2]
    tz = trans_ref[:, 2:3]

    px = points_ref[0:1, :]
    py = points_ref[1:2, :]
    pz = points_ref[2:3, :]

    d0 = px - tx
    d1 = py - ty
    d2 = pz - tz
    x0 = r00 * d0 + r10 * d1 + r20 * d2
    x1 = r01 * d0 + r11 * d1 + r21 * d2
    x2 = r02 * d0 + r12 * d1 + r22 * d2

    r = jnp.sqrt(x0 * x0 + x1 * x1 + x2 * x2) + _EPS

    xs = _signclamp(x0)
    ys = _signclamp(x1)
    zs = _signclamp(x2)

    fx = _signclamp(cx * zs + 1.0)
    fy = _signclamp(cy * zs + 1.0)
    x = xs / fx
    y = ys / fy

    lx = jnp.log2(jnp.abs(x) * isx)
    ly = jnp.log2(jnp.abs(y) * isy)
    lz = jnp.log2(jnp.abs(zs) * isz)

    A = jnp.exp2(p2 * lx) + jnp.exp2(p2 * ly)
    B = jnp.exp2(p21 * jnp.log2(A)) + jnp.exp2(p1 * lz)
    sdf = r * (1.0 - jnp.exp2(ph * jnp.log2(B)))
    out_ref[...] = jnp.clip(sdf, -_TRUNC, _TRUNC)


@functools.partial(jax.jit, static_argnames=())
def kernel(points, raw_scale, raw_exponents, raw_rotation, raw_tapering,
           translation):
    N = raw_scale.shape[0]
    M = points.shape[1]
    MB = 2048
    grid = (pl.cdiv(M, MB),)

    full = lambda shape: pl.BlockSpec(shape, lambda i: (0, 0))
    out = pl.pallas_call(
        _sdf_block_kernel,
        grid=grid,
        in_specs=[
            pl.BlockSpec((3, MB), lambda i: (0, i)),
            full((N, 3)),
            full((N, 2)),
            full((N, 4)),
            full((N, 2)),
            full((N, 3)),
        ],
        out_specs=pl.BlockSpec((N, MB), lambda i: (0, i)),
        out_shape=jax.ShapeDtypeStruct((N, M), jnp.float32),
        compiler_params=pltpu.CompilerParams(
            dimension_semantics=("arbitrary",),
        ),
    )(points, raw_scale, raw_exponents, raw_rotation, raw_tapering,
      translation)
    return out
